# baseline (speedup 1.0000x reference)
import numpy as np
import jax
import jax.numpy as jnp
from jax.experimental import pallas as pl
from jax.experimental.pallas import tpu as pltpu

NB = 8
NBD = 16


def _edge_masks(H, W):
    col = np.tile(np.arange(W), H)
    row = np.repeat(np.arange(H), W)
    return np.stack([
        (col > 0), (col < W - 1), (row > 0), (row < H - 1),
    ]).astype(np.float32)


def _interp_matrix(n_in, n_out):
    if n_in == 1:
        return np.ones((n_out, 1), np.float32)
    src = np.arange(n_out, dtype=np.float64) * (n_in - 1) / (n_out - 1)
    i0 = np.minimum(np.floor(src).astype(np.int64), n_in - 2)
    frac = src - i0
    m = np.zeros((n_out, n_in), np.float64)
    m[np.arange(n_out), i0] += 1.0 - frac
    m[np.arange(n_out), i0 + 1] += frac
    return m.astype(np.float32)


def _up2x_matrix(H0, W0):
    A = _interp_matrix(H0, 2 * H0)
    B = _interp_matrix(W0, 2 * W0)
    return np.kron(A, B).T.astype(np.float32)


def _make_body(nb, nchunk, chA, chB, C, H0, W0, H1, W1):
    bf16, f32 = jnp.bfloat16, jnp.float32
    S0, S1 = H0 * W0, H1 * W1
    M = nb * C

    def shl(v, k):
        z = jnp.zeros(v.shape[:-1] + (k,), v.dtype)
        return jnp.concatenate([v[..., k:], z], axis=-1)

    def shr(v, k):
        z = jnp.zeros(v.shape[:-1] + (k,), v.dtype)
        return jnp.concatenate([z, v[..., :-k]], axis=-1)

    def dw_conv2(z, em_ref, wcat_ref, beff_ref, W):
        zb = z.astype(bf16)
        S = z.shape[-1]
        c0 = em_ref[0].astype(bf16).reshape(1, S)
        c1 = em_ref[1].astype(bf16).reshape(1, S)
        zcat = jnp.concatenate(
            [shr(zb, 1) * c0, zb, shl(zb, 1) * c1], axis=0)
        V = jnp.dot(wcat_ref[...], zcat, preferred_element_type=f32)
        out = (V[M:2 * M]
               + shr(V[:M], W) * em_ref[2].reshape(1, S)
               + shl(V[2 * M:], W) * em_ref[3].reshape(1, S))
        return out + beff_ref[...]

    def body(x0_ref, x1_ref,
             wu1_ref, bu1_ref, emu_ref, wcu_ref, beu_ref,
             ws1_ref, bs1_ref, ems_ref, wcs_ref, bes_ref,
             upmat_ref, o_ref):
        for j in range(nchunk):
            sl = slice(j * nb, (j + 1) * nb)
            x0 = x0_ref[sl].astype(bf16).reshape(nb * chA, S0)
            y = jnp.dot(wu1_ref[...], x0, preferred_element_type=f32)
            y = jnp.maximum(y + bu1_ref[...], 0.0)
            y = jnp.maximum(dw_conv2(y, emu_ref, wcu_ref, beu_ref, W0), 0.0)
            up = jnp.dot(y.astype(bf16), upmat_ref[...],
                         preferred_element_type=f32)

            x1 = x1_ref[sl].astype(bf16).reshape(nb * chB, S1)
            z = jnp.dot(ws1_ref[...], x1, preferred_element_type=f32)
            z = jnp.maximum(z + bs1_ref[...], 0.0)
            z = jnp.maximum(dw_conv2(z, ems_ref, wcs_ref, bes_ref, W1), 0.0)

            o_ref[sl, :C] = up.reshape(nb, C, H1, W1)
            o_ref[sl, C:] = z.reshape(nb, C, H1, W1)

    return body


def kernel(x0, x1, up_w1, up_b1, up_wdw, up_bdw, up_w2, up_b2,
           sh_w1, sh_b1, sh_wdw, sh_bdw, sh_w2, sh_b2):
    N, chA, H0, W0 = x0.shape
    _, chB, H1, W1 = x1.shape
    C = up_w1.shape[0]
    S0, S1 = H0 * W0, H1 * W1
    nbd = NBD if N % NBD == 0 else 1
    nb = NB if nbd % NB == 0 else 1
    nchunk = nbd // nb
    bf16 = jnp.bfloat16

    eye = jnp.eye(nb, dtype=jnp.float32)

    def bd(w):
        return jnp.kron(eye, w)

    def tl(b):
        return jnp.tile(b, (nb, 1))

    def fold_dw_conv2(w2, wdw, bdw, b2):
        taps = wdw.reshape(9, C)
        rows = [jnp.concatenate([bd(w2 * taps[3 * r + c][None, :])
                                 for c in range(3)], axis=1)
                for r in range(3)]
        wcat = jnp.concatenate(rows, axis=0).astype(bf16)
        beff = tl(jnp.dot(w2, bdw) + b2)
        return wcat, beff

    wu1 = bd(up_w1).astype(bf16)
    ws1 = bd(sh_w1).astype(bf16)
    bu1, bs1 = tl(up_b1), tl(sh_b1)
    wcu, beu = fold_dw_conv2(up_w2, up_wdw, up_bdw, up_b2)
    wcs, bes = fold_dw_conv2(sh_w2, sh_wdw, sh_bdw, sh_b2)

    emu = jnp.asarray(_edge_masks(H0, W0))
    ems = jnp.asarray(_edge_masks(H1, W1))
    upmat = jnp.asarray(_up2x_matrix(H0, W0)).astype(bf16)

    body = _make_body(nb, nchunk, chA, chB, C, H0, W0, H1, W1)

    def cspec(arr):
        nd = arr.ndim
        return pl.BlockSpec(arr.shape, lambda b, _nd=nd: (0,) * _nd)

    out = pl.pallas_call(
        body,
        out_shape=jax.ShapeDtypeStruct((N, 2 * C, H1, W1), jnp.float32),
        grid=(N // nbd,),
        in_specs=[
            pl.BlockSpec((nbd, chA, H0, W0), lambda b: (b, 0, 0, 0)),
            pl.BlockSpec((nbd, chB, H1, W1), lambda b: (b, 0, 0, 0)),
            cspec(wu1), cspec(bu1), cspec(emu), cspec(wcu), cspec(beu),
            cspec(ws1), cspec(bs1), cspec(ems), cspec(wcs), cspec(bes),
            cspec(upmat),
        ],
        out_specs=pl.BlockSpec((nbd, 2 * C, H1, W1), lambda b: (b, 0, 0, 0)),
        compiler_params=pltpu.CompilerParams(
            dimension_semantics=("parallel",),
            vmem_limit_bytes=100 * 1024 * 1024,
        ),
    )(x0, x1, wu1, bu1, emu, wcu, beu,
      ws1, bs1, ems, wcs, bes, upmat)

    return out


# speedup vs baseline: 2.4034x; 2.4034x over previous
import numpy as np
import jax
import jax.numpy as jnp
from jax.experimental import pallas as pl
from jax.experimental.pallas import tpu as pltpu

NB = 8
NBD = 16


def _edge_masks(H, W):
    col = np.tile(np.arange(W), H)
    row = np.repeat(np.arange(H), W)
    return np.stack([
        (col > 0), (col < W - 1), (row > 0), (row < H - 1),
    ]).astype(np.float32)


def _interp_matrix(n_in, n_out):
    if n_in == 1:
        return np.ones((n_out, 1), np.float32)
    src = np.arange(n_out, dtype=np.float64) * (n_in - 1) / (n_out - 1)
    i0 = np.minimum(np.floor(src).astype(np.int64), n_in - 2)
    frac = src - i0
    m = np.zeros((n_out, n_in), np.float64)
    m[np.arange(n_out), i0] += 1.0 - frac
    m[np.arange(n_out), i0 + 1] += frac
    return m.astype(np.float32)


def _up2x_matrix(H0, W0):
    A = _interp_matrix(H0, 2 * H0)
    B = _interp_matrix(W0, 2 * W0)
    return np.kron(A, B).T.astype(np.float32)


def _make_body(nb, nchunk, chA, chB, C, H0, W0, H1, W1):
    bf16, f32 = jnp.bfloat16, jnp.float32
    S0, S1 = H0 * W0, H1 * W1
    M = nb * C

    def shl(v, k):
        z = jnp.zeros(v.shape[:-1] + (k,), v.dtype)
        return jnp.concatenate([v[..., k:], z], axis=-1)

    def shr(v, k):
        z = jnp.zeros(v.shape[:-1] + (k,), v.dtype)
        return jnp.concatenate([z, v[..., :-k]], axis=-1)

    def dw_conv2(z, em_ref, wcat_ref, beff_ref, W):
        zb = z.astype(bf16)
        S = z.shape[-1]
        c0 = em_ref[0].astype(bf16).reshape(1, S)
        c1 = em_ref[1].astype(bf16).reshape(1, S)
        zcat = jnp.concatenate(
            [shr(zb, 1) * c0, zb, shl(zb, 1) * c1], axis=0)
        V = jnp.dot(wcat_ref[...], zcat, preferred_element_type=f32)
        out = (V[M:2 * M]
               + shr(V[:M], W) * em_ref[2].reshape(1, S)
               + shl(V[2 * M:], W) * em_ref[3].reshape(1, S))
        return out + beff_ref[...]

    def body(x0_ref, x1_ref,
             wu1_ref, bu1_ref, emu_ref, wcu_ref, beu_ref,
             ws1_ref, bs1_ref, ems_ref, wcs_ref, bes_ref,
             upmat_ref, o_ref):
        for j in range(nchunk):
            sl = slice(j * nb, (j + 1) * nb)
            x0 = x0_ref[sl].reshape(nb * chA, S0)
            y = jnp.dot(wu1_ref[...], x0, preferred_element_type=f32)
            y = jnp.maximum(y + bu1_ref[...], 0.0)
            y = jnp.maximum(dw_conv2(y, emu_ref, wcu_ref, beu_ref, W0), 0.0)
            up = jnp.dot(y.astype(bf16), upmat_ref[...],
                         preferred_element_type=f32)

            x1 = x1_ref[sl].reshape(nb * chB, S1)
            z = jnp.dot(ws1_ref[...], x1, preferred_element_type=f32)
            z = jnp.maximum(z + bs1_ref[...], 0.0)
            z = jnp.maximum(dw_conv2(z, ems_ref, wcs_ref, bes_ref, W1), 0.0)

            o_ref[sl, :C, :] = up.reshape(nb, C, S1)
            o_ref[sl, C:, :] = z.reshape(nb, C, S1)

    return body


def kernel(x0, x1, up_w1, up_b1, up_wdw, up_bdw, up_w2, up_b2,
           sh_w1, sh_b1, sh_wdw, sh_bdw, sh_w2, sh_b2):
    N, chA, H0, W0 = x0.shape
    _, chB, H1, W1 = x1.shape
    C = up_w1.shape[0]
    S0, S1 = H0 * W0, H1 * W1
    nbd = NBD if N % NBD == 0 else 1
    nb = NB if nbd % NB == 0 else 1
    nchunk = nbd // nb
    bf16 = jnp.bfloat16

    x0f = x0.reshape(N, chA, S0).astype(bf16)
    x1f = x1.reshape(N, chB, S1).astype(bf16)

    eye = jnp.eye(nb, dtype=jnp.float32)

    def bd(w):
        return jnp.kron(eye, w)

    def tl(b):
        return jnp.tile(b, (nb, 1))

    def fold_dw_conv2(w2, wdw, bdw, b2):
        taps = wdw.reshape(9, C)
        rows = [jnp.concatenate([bd(w2 * taps[3 * r + c][None, :])
                                 for c in range(3)], axis=1)
                for r in range(3)]
        wcat = jnp.concatenate(rows, axis=0).astype(bf16)
        beff = tl(jnp.dot(w2, bdw) + b2)
        return wcat, beff

    wu1 = bd(up_w1).astype(bf16)
    ws1 = bd(sh_w1).astype(bf16)
    bu1, bs1 = tl(up_b1), tl(sh_b1)
    wcu, beu = fold_dw_conv2(up_w2, up_wdw, up_bdw, up_b2)
    wcs, bes = fold_dw_conv2(sh_w2, sh_wdw, sh_bdw, sh_b2)

    emu = jnp.asarray(_edge_masks(H0, W0))
    ems = jnp.asarray(_edge_masks(H1, W1))
    upmat = jnp.asarray(_up2x_matrix(H0, W0)).astype(bf16)

    body = _make_body(nb, nchunk, chA, chB, C, H0, W0, H1, W1)

    def cspec(arr):
        nd = arr.ndim
        return pl.BlockSpec(arr.shape, lambda b, _nd=nd: (0,) * _nd)

    out = pl.pallas_call(
        body,
        out_shape=jax.ShapeDtypeStruct((N, 2 * C, S1), jnp.float32),
        grid=(N // nbd,),
        in_specs=[
            pl.BlockSpec((nbd, chA, S0), lambda b: (b, 0, 0)),
            pl.BlockSpec((nbd, chB, S1), lambda b: (b, 0, 0)),
            cspec(wu1), cspec(bu1), cspec(emu), cspec(wcu), cspec(beu),
            cspec(ws1), cspec(bs1), cspec(ems), cspec(wcs), cspec(bes),
            cspec(upmat),
        ],
        out_specs=pl.BlockSpec((nbd, 2 * C, S1), lambda b: (b, 0, 0)),
        compiler_params=pltpu.CompilerParams(
            dimension_semantics=("parallel",),
            vmem_limit_bytes=100 * 1024 * 1024,
        ),
    )(x0f, x1f, wu1, bu1, emu, wcu, beu,
      ws1, bs1, ems, wcs, bes, upmat)

    return out.reshape(N, 2 * C, H1, W1)
